# baseline (speedup 1.0000x reference)
import functools
import math

import jax
import jax.numpy as jnp
from jax import lax
from jax.experimental import pallas as pl
from jax.experimental.pallas import tpu as pltpu
from jax.experimental.pallas import tpu_sc as plsc

B, C, H, W = 2, 96, 56, 56
K = 9
R = 2
GROUPS = 4
N = H * W
M = (H // R) * (W // R)
BN = 784
NBLK = (B * N) // BN

NC, NS = 2, 16
NW = NC * NS
P_PAD = 6400
PPW = P_PAD // NW
CHP = 40
NCHUNK = PPW // CHP

CP = 128
_BIG = 3.0e38


def _knn_body(xb_ref, ycm_ref, idx_ref):
    blk = pl.program_id(0)
    boff = (blk // (N // BN)) * M
    xb = xb_ref[...]
    ycm = ycm_ref[0]

    xnorm = jnp.sqrt(jnp.sum(xb * xb, axis=1, keepdims=True))
    xn = xb / jnp.maximum(xnorm, 1e-12)
    ynorm = jnp.sqrt(jnp.sum(ycm * ycm, axis=0, keepdims=True))
    yn = ycm / jnp.maximum(ynorm, 1e-12)
    q = jnp.sum(yn * yn, axis=0, keepdims=True)

    dist = q - 2.0 * jnp.dot(xn, yn, preferred_element_type=jnp.float32)

    lane = lax.broadcasted_iota(jnp.int32, (BN, M), 1)
    kiota = lax.broadcasted_iota(jnp.int32, (BN, K), 1)
    idxacc = jnp.zeros((BN, K), jnp.int32)
    for t in range(K):
        v = jnp.min(dist, axis=1, keepdims=True)
        idx = jnp.min(jnp.where(dist <= v, lane, M), axis=1, keepdims=True)
        idxacc = idxacc + jnp.where(kiota == t, idx + boff, 0)
        dist = jnp.where(lane == idx, _BIG, dist)
    idx_ref[...] = idxacc


def _gather_max_body(y_hbm, idx_hbm, g_hbm, idx_v, rows_v, out_v, sem):
    wid = lax.axis_index("s") * NC + lax.axis_index("c")
    for ci in range(NCHUNK):
        base_p = wid * PPW + ci * CHP
        pltpu.sync_copy(idx_hbm.at[pl.ds(base_p * K, CHP * K)], idx_v)
        pltpu.async_copy(y_hbm.at[idx_v], rows_v, sem).wait()

        def point_body(p, carry):
            for l in range(CP // 16):
                sl = pl.ds(l * 16, 16)
                m = rows_v[p * K, sl]
                for r in range(1, K):
                    m = jnp.maximum(m, rows_v[p * K + r, sl])
                out_v[p, sl] = m
            return carry

        lax.fori_loop(0, CHP, point_body, 0)
        pltpu.sync_copy(out_v, g_hbm.at[pl.ds(base_p, CHP)])


def _conv_body(xb_ref, g_ref, wd_ref, wb_ref, beta_ref, out_ref):
    xb = xb_ref[...]
    acc = g_ref[...]
    o = (jnp.dot(xb, wd_ref[...], preferred_element_type=jnp.float32)
         + jnp.dot(acc, wb_ref[...], preferred_element_type=jnp.float32)
         + beta_ref[0:1, :])
    out_ref[...] = o * 0.5 * (1.0 + lax.erf(o * (1.0 / math.sqrt(2.0))))


@jax.jit
def kernel(x, conv_w, conv_b, bn_w, bn_b):
    xf = x.reshape(B, C, N)
    x_nc = xf.transpose(0, 2, 1).reshape(B * N, C)
    y = x.reshape(B, C, H // R, R, W // R, R).mean(axis=(3, 5))
    y_cm = y.reshape(B, C, M)
    y_all = y_cm.transpose(0, 2, 1).reshape(B * M, C)
    y_pad = jnp.pad(y_all, ((0, 0), (0, CP - C)))

    nn_idx = pl.pallas_call(
        _knn_body,
        grid=(NBLK,),
        in_specs=[
            pl.BlockSpec((BN, C), lambda i: (i, 0)),
            pl.BlockSpec((1, C, M), lambda i: (i // (N // BN), 0, 0)),
        ],
        out_specs=pl.BlockSpec((BN, K), lambda i: (i, 0)),
        out_shape=jax.ShapeDtypeStruct((B * N, K), jnp.int32),
    )(x_nc, y_cm)

    idx_flat = jnp.concatenate(
        [nn_idx, jnp.zeros((P_PAD - B * N, K), jnp.int32)]).reshape(-1)

    mesh = plsc.VectorSubcoreMesh(core_axis_name="c", subcore_axis_name="s")
    gather_max = pl.kernel(
        _gather_max_body,
        out_type=jax.ShapeDtypeStruct((P_PAD, CP), jnp.float32),
        mesh=mesh,
        scratch_types=[
            pltpu.VMEM((CHP * K,), jnp.int32),
            pltpu.VMEM((CHP * K, CP), jnp.float32),
            pltpu.VMEM((CHP, CP), jnp.float32),
            pltpu.SemaphoreType.DMA,
        ],
    )
    g = gather_max(y_pad, idx_flat)

    w2 = conv_w[:, :, 0, 0]
    gout = C // GROUPS
    wa = jnp.zeros((C, C), jnp.float32)
    wb = jnp.zeros((C, C), jnp.float32)
    for gi in range(GROUPS):
        sl = slice(gi * gout, (gi + 1) * gout)
        blk = w2[sl, :]
        wa = wa.at[sl, sl].set(blk[:, 0::2].T)
        wb = wb.at[sl, sl].set(blk[:, 1::2].T)
    alpha = bn_w * (1.0 / math.sqrt(1.0 + 1e-5))
    wa = wa * alpha[None, :]
    wb = wb * alpha[None, :]
    wd = wa - wb
    wb_pad = jnp.pad(wb, ((0, CP - C), (0, 0)))
    beta = conv_b * alpha + bn_b
    beta8 = jnp.broadcast_to(beta[None, :], (8, C))

    out_flat = pl.pallas_call(
        _conv_body,
        grid=(NBLK,),
        in_specs=[
            pl.BlockSpec((BN, C), lambda i: (i, 0)),
            pl.BlockSpec((BN, CP), lambda i: (i, 0)),
            pl.BlockSpec((C, C), lambda i: (0, 0)),
            pl.BlockSpec((CP, C), lambda i: (0, 0)),
            pl.BlockSpec((8, C), lambda i: (0, 0)),
        ],
        out_specs=pl.BlockSpec((BN, C), lambda i: (i, 0)),
        out_shape=jax.ShapeDtypeStruct((B * N, C), jnp.float32),
    )(x_nc, g, wd, wb_pad, beta8)

    return out_flat.reshape(B, N, C).transpose(0, 2, 1).reshape(B, C, H, W)


# speedup vs baseline: 1.0677x; 1.0677x over previous
import functools
import math

import jax
import jax.numpy as jnp
from jax import lax
from jax.experimental import pallas as pl
from jax.experimental.pallas import tpu as pltpu
from jax.experimental.pallas import tpu_sc as plsc

B, C, H, W = 2, 96, 56, 56
K = 9
R = 2
GROUPS = 4
N = H * W
M = (H // R) * (W // R)
BN = 784
NBLK = (B * N) // BN

NC, NS = 2, 16
NW = NC * NS
P_PAD = 6400
PPW = P_PAD // NW
CHP = 40
NCHUNK = PPW // CHP

CP = 128
_BIG = 3.0e38


def _knn_body(xb_ref, ycm_ref, idx_ref):
    blk = pl.program_id(0)
    boff = (blk // (N // BN)) * M
    xb = xb_ref[...]
    ycm = ycm_ref[0]

    xnorm = jnp.sqrt(jnp.sum(xb * xb, axis=1, keepdims=True))
    xn = xb / jnp.maximum(xnorm, 1e-12)
    ynorm = jnp.sqrt(jnp.sum(ycm * ycm, axis=0, keepdims=True))
    yn = ycm / jnp.maximum(ynorm, 1e-12)
    q = jnp.sum(yn * yn, axis=0, keepdims=True)

    dist = q - 2.0 * jnp.dot(xn, yn, preferred_element_type=jnp.float32)

    lane = lax.broadcasted_iota(jnp.int32, (BN, M), 1)
    kiota = lax.broadcasted_iota(jnp.int32, (BN, K), 1)
    idxacc = jnp.zeros((BN, K), jnp.int32)
    for t in range(K):
        v = jnp.min(dist, axis=1, keepdims=True)
        idx = jnp.min(jnp.where(dist <= v, lane, M), axis=1, keepdims=True)
        idxacc = idxacc + jnp.where(kiota == t, idx + boff, 0)
        dist = jnp.where(lane == idx, _BIG, dist)
    idx_ref[...] = idxacc


def _gather_max_body(y_hbm, idx_hbm, g_hbm, idx_v, rows0, rows1, out0, out1,
                     gs0, gs1, os0, os1):
    wid = lax.axis_index("s") * NC + lax.axis_index("c")
    rows = (rows0, rows1)
    outs = (out0, out1)
    gsem = (gs0, gs1)
    osem = (os0, os1)

    pltpu.sync_copy(idx_hbm.at[pl.ds(wid * PPW * K, PPW * K)], idx_v)

    def start_gather(c):
        return pltpu.async_copy(y_hbm.at[idx_v.at[pl.ds(c * CHP * K, CHP * K)]],
                                rows[c % 2], gsem[c % 2])

    gather_h = {0: start_gather(0)}
    out_h = {}
    for c in range(NCHUNK):
        if c + 1 < NCHUNK:
            gather_h[c + 1] = start_gather(c + 1)
        gather_h[c].wait()
        if c >= 2:
            out_h[c - 2].wait()
        rv = rows[c % 2]
        ov = outs[c % 2]

        def point_body(p, carry):
            for l in range(C // 16):
                sl = pl.ds(l * 16, 16)
                m = rv[p * K, sl]
                for r in range(1, K):
                    m = jnp.maximum(m, rv[p * K + r, sl])
                ov[p, sl] = m
            return carry

        lax.fori_loop(0, CHP, point_body, 0)
        base_p = wid * PPW + c * CHP
        out_h[c] = pltpu.async_copy(ov, g_hbm.at[pl.ds(base_p, CHP)],
                                    osem[c % 2])
    out_h[NCHUNK - 2].wait()
    out_h[NCHUNK - 1].wait()


def _conv_body(xb_ref, g_ref, wd_ref, wb_ref, beta_ref, out_ref):
    xb = xb_ref[...]
    acc = g_ref[...]
    o = (jnp.dot(xb, wd_ref[...], preferred_element_type=jnp.float32)
         + jnp.dot(acc, wb_ref[...], preferred_element_type=jnp.float32)
         + beta_ref[0:1, :])
    out_ref[...] = o * 0.5 * (1.0 + lax.erf(o * (1.0 / math.sqrt(2.0))))


@jax.jit
def kernel(x, conv_w, conv_b, bn_w, bn_b):
    xf = x.reshape(B, C, N)
    x_nc = xf.transpose(0, 2, 1).reshape(B * N, C)
    y = x.reshape(B, C, H // R, R, W // R, R).mean(axis=(3, 5))
    y_cm = y.reshape(B, C, M)
    y_all = y_cm.transpose(0, 2, 1).reshape(B * M, C)
    y_pad = jnp.pad(y_all, ((0, 0), (0, CP - C)))

    nn_idx = pl.pallas_call(
        _knn_body,
        grid=(NBLK,),
        in_specs=[
            pl.BlockSpec((BN, C), lambda i: (i, 0)),
            pl.BlockSpec((1, C, M), lambda i: (i // (N // BN), 0, 0)),
        ],
        out_specs=pl.BlockSpec((BN, K), lambda i: (i, 0)),
        out_shape=jax.ShapeDtypeStruct((B * N, K), jnp.int32),
    )(x_nc, y_cm)

    idx_flat = jnp.concatenate(
        [nn_idx, jnp.zeros((P_PAD - B * N, K), jnp.int32)]).reshape(-1)

    mesh = plsc.VectorSubcoreMesh(core_axis_name="c", subcore_axis_name="s")
    gather_max = pl.kernel(
        _gather_max_body,
        out_type=jax.ShapeDtypeStruct((P_PAD, C), jnp.float32),
        mesh=mesh,
        scratch_types=[
            pltpu.VMEM((PPW * K,), jnp.int32),
            pltpu.VMEM((CHP * K, CP), jnp.float32),
            pltpu.VMEM((CHP * K, CP), jnp.float32),
            pltpu.VMEM((CHP, C), jnp.float32),
            pltpu.VMEM((CHP, C), jnp.float32),
            pltpu.SemaphoreType.DMA,
            pltpu.SemaphoreType.DMA,
            pltpu.SemaphoreType.DMA,
            pltpu.SemaphoreType.DMA,
        ],
    )
    g = gather_max(y_pad, idx_flat)

    w2 = conv_w[:, :, 0, 0]
    gout = C // GROUPS
    wa = jnp.zeros((C, C), jnp.float32)
    wb = jnp.zeros((C, C), jnp.float32)
    for gi in range(GROUPS):
        sl = slice(gi * gout, (gi + 1) * gout)
        blk = w2[sl, :]
        wa = wa.at[sl, sl].set(blk[:, 0::2].T)
        wb = wb.at[sl, sl].set(blk[:, 1::2].T)
    alpha = bn_w * (1.0 / math.sqrt(1.0 + 1e-5))
    wa = wa * alpha[None, :]
    wb = wb * alpha[None, :]
    wd = wa - wb
    beta = conv_b * alpha + bn_b
    beta8 = jnp.broadcast_to(beta[None, :], (8, C))

    out_flat = pl.pallas_call(
        _conv_body,
        grid=(NBLK,),
        in_specs=[
            pl.BlockSpec((BN, C), lambda i: (i, 0)),
            pl.BlockSpec((BN, C), lambda i: (i, 0)),
            pl.BlockSpec((C, C), lambda i: (0, 0)),
            pl.BlockSpec((C, C), lambda i: (0, 0)),
            pl.BlockSpec((8, C), lambda i: (0, 0)),
        ],
        out_specs=pl.BlockSpec((BN, C), lambda i: (i, 0)),
        out_shape=jax.ShapeDtypeStruct((B * N, C), jnp.float32),
    )(x_nc, g, wd, wb, beta8)

    return out_flat.reshape(B, N, C).transpose(0, 2, 1).reshape(B, C, H, W)


# speedup vs baseline: 1.7991x; 1.6851x over previous
import math

import jax
import jax.numpy as jnp
from jax import lax
from jax.experimental import pallas as pl
from jax.experimental.pallas import tpu as pltpu
from jax.experimental.pallas import tpu_sc as plsc

B, C, H, W = 2, 96, 56, 56
K = 9
R = 2
GROUPS = 4
N = H * W
M = (H // R) * (W // R)
BN = 784
NBLK = N // BN

NC, NS = 2, 16
NW = NC * NS
PPW = 104
CHUNKS = (40, 40, 24)
CP = 128

_BIG = 3.0e38


def _knn_body(xb_ref, ycm_ref, idx_ref):
    xb = xb_ref[...]
    ycm = ycm_ref[...]

    xnorm = jnp.sqrt(jnp.sum(xb * xb, axis=1, keepdims=True))
    xn = xb / jnp.maximum(xnorm, 1e-12)
    ynorm = jnp.sqrt(jnp.sum(ycm * ycm, axis=0, keepdims=True))
    yn = ycm / jnp.maximum(ynorm, 1e-12)
    q = jnp.sum(yn * yn, axis=0, keepdims=True)

    dist = q - 2.0 * jnp.dot(xn, yn, preferred_element_type=jnp.float32)

    lane = lax.broadcasted_iota(jnp.int32, (BN, M), 1).astype(jnp.float32)
    kiota = lax.broadcasted_iota(jnp.int32, (BN, K), 1)
    idxacc = jnp.zeros((BN, K), jnp.float32)
    for t in range(K):
        v = jnp.min(dist, axis=1, keepdims=True)
        sel = dist <= v
        idx = jnp.min(jnp.where(sel, lane, 1e9), axis=1, keepdims=True)
        idxacc = idxacc + jnp.where(kiota == t, idx, 0.0)
        dist = jnp.where(sel, _BIG, dist)
    idx_ref[...] = idxacc.astype(jnp.int32)


def _gather_max_body(y_hbm, idx_hbm, g_hbm, idx_v, rows0, rows1, out0, out1,
                     gs0, gs1, os0, os1):
    wid = lax.axis_index("s") * NC + lax.axis_index("c")
    start_p = (wid * 98) // 8 * 8
    rows = (rows0, rows1)
    outs = (out0, out1)
    gsem = (gs0, gs1)
    osem = (os0, os1)
    bases = []
    off = 0
    for n in CHUNKS:
        bases.append(off)
        off += n

    pltpu.sync_copy(idx_hbm.at[pl.ds(start_p * K, PPW * K)], idx_v)

    def start_gather(c):
        n = CHUNKS[c]
        return pltpu.async_copy(
            y_hbm.at[idx_v.at[pl.ds(bases[c] * K, n * K)]],
            rows[c % 2].at[pl.ds(0, n * K)], gsem[c % 2])

    gather_h = {0: start_gather(0)}
    out_h = {}
    for c, n in enumerate(CHUNKS):
        if c + 1 < len(CHUNKS):
            gather_h[c + 1] = start_gather(c + 1)
        gather_h[c].wait()
        if c >= 2:
            out_h[c - 2].wait()
        rv = rows[c % 2]
        ov = outs[c % 2]

        def point_body(p, carry):
            for l in range(C // 16):
                sl = pl.ds(l * 16, 16)
                m = rv[p * K, sl]
                for r in range(1, K):
                    m = jnp.maximum(m, rv[p * K + r, sl])
                ov[p, sl] = m
            return carry

        lax.fori_loop(0, n, point_body, 0)
        out_h[c] = pltpu.async_copy(
            ov.at[pl.ds(0, n)], g_hbm.at[pl.ds(start_p + bases[c], n)],
            osem[c % 2])
    out_h[len(CHUNKS) - 2].wait()
    out_h[len(CHUNKS) - 1].wait()


def _conv_body(xb_ref, g_ref, wd_ref, wb_ref, beta_ref, out_ref):
    xb = xb_ref[...]
    acc = g_ref[...]
    o = (jnp.dot(xb, wd_ref[...], preferred_element_type=jnp.float32)
         + jnp.dot(acc, wb_ref[...], preferred_element_type=jnp.float32)
         + beta_ref[0:1, :])
    out_ref[...] = o * 0.5 * (1.0 + lax.erf(o * (1.0 / math.sqrt(2.0))))


@jax.jit
def kernel(x, conv_w, conv_b, bn_w, bn_b):
    x_nc = x.reshape(B, C, N).transpose(0, 2, 1)
    y = x.reshape(B, C, H // R, R, W // R, R).mean(axis=(3, 5))
    y_cm = y.reshape(B, C, M)
    y_pad = jnp.pad(y_cm.transpose(0, 2, 1), ((0, 0), (0, 0), (0, CP - C)))

    w2 = conv_w[:, :, 0, 0]
    gout = C // GROUPS
    wa = jnp.zeros((C, C), jnp.float32)
    wb = jnp.zeros((C, C), jnp.float32)
    for gi in range(GROUPS):
        sl = slice(gi * gout, (gi + 1) * gout)
        blk = w2[sl, :]
        wa = wa.at[sl, sl].set(blk[:, 0::2].T)
        wb = wb.at[sl, sl].set(blk[:, 1::2].T)
    alpha = bn_w * (1.0 / math.sqrt(1.0 + 1e-5))
    wa = wa * alpha[None, :]
    wb = wb * alpha[None, :]
    wd = wa - wb
    beta = conv_b * alpha + bn_b
    beta8 = jnp.broadcast_to(beta[None, :], (8, C))

    mesh = plsc.VectorSubcoreMesh(core_axis_name="c", subcore_axis_name="s")
    gather_max = pl.kernel(
        _gather_max_body,
        out_type=jax.ShapeDtypeStruct((N, C), jnp.float32),
        mesh=mesh,
        scratch_types=[
            pltpu.VMEM((PPW * K,), jnp.int32),
            pltpu.VMEM((CHUNKS[0] * K, CP), jnp.float32),
            pltpu.VMEM((CHUNKS[0] * K, CP), jnp.float32),
            pltpu.VMEM((CHUNKS[0], C), jnp.float32),
            pltpu.VMEM((CHUNKS[0], C), jnp.float32),
            pltpu.SemaphoreType.DMA,
            pltpu.SemaphoreType.DMA,
            pltpu.SemaphoreType.DMA,
            pltpu.SemaphoreType.DMA,
        ],
    )

    outs = []
    for b in range(B):
        nn_idx = pl.pallas_call(
            _knn_body,
            grid=(NBLK,),
            in_specs=[
                pl.BlockSpec((BN, C), lambda i: (i, 0)),
                pl.BlockSpec((C, M), lambda i: (0, 0)),
            ],
            out_specs=pl.BlockSpec((BN, K), lambda i: (i, 0)),
            out_shape=jax.ShapeDtypeStruct((N, K), jnp.int32),
        )(x_nc[b], y_cm[b])

        g = gather_max(y_pad[b], nn_idx.reshape(-1))

        out_b = pl.pallas_call(
            _conv_body,
            grid=(NBLK,),
            in_specs=[
                pl.BlockSpec((BN, C), lambda i: (i, 0)),
                pl.BlockSpec((BN, C), lambda i: (i, 0)),
                pl.BlockSpec((C, C), lambda i: (0, 0)),
                pl.BlockSpec((C, C), lambda i: (0, 0)),
                pl.BlockSpec((8, C), lambda i: (0, 0)),
            ],
            out_specs=pl.BlockSpec((BN, C), lambda i: (i, 0)),
            out_shape=jax.ShapeDtypeStruct((N, C), jnp.float32),
        )(x_nc[b], g, wd, wb, beta8)
        outs.append(out_b)

    out = jnp.stack(outs)
    return out.transpose(0, 2, 1).reshape(B, C, H, W)
